# baseline (speedup 1.0000x reference)
import functools

import jax
import jax.numpy as jnp
from jax import lax
from jax.experimental import pallas as pl
from jax.experimental.pallas import tpu as pltpu
from jax.experimental.pallas import tpu_sc as plsc

_NC = 2
_NS = 16
_NW = _NC * _NS
_L = 16
_BLK = 128
_CB = 22
_CHUNK = _CB * _BLK
_R = 3


@functools.partial(jax.jit, static_argnames=("interpret",))
def _sc_partials(probs, edges, batch, interpret=False):
    N = probs.shape[0]
    E = edges.shape[1]
    nblk = E // _BLK
    mainb = nblk // _NW
    rem = nblk - mainb * _NW
    nchunk = mainb // _CB
    nvec = N // _L
    vpw = -(-nvec // _NW)

    mesh = plsc.VectorSubcoreMesh(
        core_axis_name="c", subcore_axis_name="s",
        num_cores=_NC, num_subcores=_NS)

    @functools.partial(
        pl.kernel,
        out_type=(
            jax.ShapeDtypeStruct((_NW, _L), jnp.float32),
            jax.ShapeDtypeStruct((_NW, _L), jnp.float32),
            jax.ShapeDtypeStruct((_NW, _L), jnp.int32),
        ),
        mesh=mesh,
        interpret=interpret,
        compiler_params=pltpu.CompilerParams(needs_layout_passes=False),
        scratch_types=[
            pltpu.VMEM((N,), jnp.float32),
            pltpu.VMEM_SHARED((N,), jnp.float32),
            *[pltpu.VMEM((2, _CHUNK), jnp.int32) for _ in range(_R)],
            pltpu.VMEM((2, _BLK), jnp.int32),
            pltpu.VMEM((vpw * _L,), jnp.int32),
            pltpu.VMEM((_L,), jnp.float32),
            pltpu.VMEM((_L,), jnp.float32),
            pltpu.VMEM((_L,), jnp.int32),
            pltpu.SemaphoreType.DMA,
            pltpu.SemaphoreType.DMA,
            pltpu.SemaphoreType.DMA,
            *[pltpu.SemaphoreType.DMA for _ in range(_R)],
        ],
    )
    def sc_kernel(probs_hbm, edges_hbm, batch_hbm,
                  edge_out, psum_out, bmax_out,
                  table_v, table_s, *rest):
        ebufs = rest[:_R]
        xbuf, batch_v, oe_v, op_v, om_v, sem_t, sem_b, sem_x = rest[_R:_R + 8]
        esems = rest[_R + 8:]
        wid = lax.axis_index("s") * _NC + lax.axis_index("c")

        has_extra = wid < rem
        base_b = jnp.where(has_extra, wid * (mainb + 1), rem + wid * mainb)
        mstart = base_b + jnp.where(has_extra, 1, 0)

        def edge_cp(chunk, b):
            off = (mstart + chunk * _CB) * _BLK
            return pltpu.make_async_copy(
                edges_hbm.at[:, pl.ds(off, _CHUNK)], ebufs[b], esems[b])

        def edge_block(buf, off, a):
            ir = buf[0, pl.ds(off, _L)]
            ic = buf[1, pl.ds(off, _L)]
            pr = plsc.load_gather(table_v, [ir])
            pc = plsc.load_gather(table_v, [ic])
            return a + jnp.where(ir != ic, (1.0 - pr) * (1.0 - pc), 0.0)

        bstart = jnp.minimum(wid * vpw, nvec - vpw) * _L
        cp_b = pltpu.make_async_copy(
            batch_hbm.at[pl.ds(bstart, vpw * _L)], batch_v, sem_b)
        cp_b.start()
        for b in range(_R):
            edge_cp(b, b).start()

        @pl.when(lax.axis_index("s") == 0)
        def _():
            pltpu.sync_copy(probs_hbm, table_s)
        plsc.subcore_barrier()
        pltpu.sync_copy(table_s, table_v)
        pvec0 = wid * vpw
        ptrip = jnp.minimum(vpw, nvec - pvec0)

        @pl.loop(0, ptrip, init_carry=jnp.zeros((_L,), jnp.float32))
        def psum(i, acc):
            return acc + table_v[pl.ds((pvec0 + i) * _L, _L)]
        op_v[...] = psum

        cp_b.wait()

        @pl.loop(0, vpw, init_carry=jnp.zeros((_L,), jnp.int32))
        def bmax(i, acc):
            return jnp.maximum(acc, batch_v[pl.ds(i * _L, _L)])
        om_v[...] = bmax

        oe_v[...] = jnp.zeros((_L,), jnp.float32)

        @pl.when(has_extra)
        def _():
            cpx = pltpu.make_async_copy(
                edges_hbm.at[:, pl.ds(base_b * _BLK, _BLK)], xbuf, sem_x)
            cpx.start()
            cpx.wait()

            @pl.loop(0, _BLK, step=_L, init_carry=jnp.zeros((_L,), jnp.float32))
            def xacc(off, a):
                return edge_block(xbuf, off, a)
            oe_v[...] = xacc

        def chunk_compute(buf, acc):
            @plsc.parallel_loop(0, _CHUNK, step=_L, unroll=8, carry=acc)
            def inner(off, a):
                return edge_block(buf, off, a)
            return inner

        rounds = nchunk // _R

        @pl.loop(0, rounds, init_carry=oe_v[...])
        def eloop(g, acc):
            for b in range(_R):
                chunk = g * _R + b
                edge_cp(chunk, b).wait()
                acc = chunk_compute(ebufs[b], acc)

                refill = chunk + _R - 1
                cond = refill < nchunk
                if b == 0:
                    cond = jnp.logical_and(g > 0, cond)

                @pl.when(cond)
                def _():
                    edge_cp(refill, (b - 1) % _R).start()
            return acc

        acc = eloop
        for j in range(rounds * _R, nchunk):
            edge_cp(j, j % _R).wait()
            acc = chunk_compute(ebufs[j % _R], acc)
        oe_v[...] = acc

        pltpu.sync_copy(oe_v, edge_out.at[wid])
        pltpu.sync_copy(op_v, psum_out.at[wid])
        pltpu.sync_copy(om_v, bmax_out.at[wid])

    return sc_kernel(probs, edges, batch)


def kernel(probs, edge_index, batch, penalty_coefficient):
    e_part, p_part, m_part = _sc_partials(probs, edge_index, batch)
    num_graphs = jnp.max(m_part) + 1
    expected_distance = jnp.sum(e_part) / num_graphs
    expected_weight = jnp.sum(p_part) / num_graphs
    loss = penalty_coefficient * expected_distance + expected_weight
    return (loss, expected_weight, expected_distance)


# speedup vs baseline: 1.0700x; 1.0700x over previous
import functools

import jax
import jax.numpy as jnp
from jax import lax
from jax.experimental import pallas as pl
from jax.experimental.pallas import tpu as pltpu
from jax.experimental.pallas import tpu_sc as plsc

_NC = 2
_NS = 16
_NW = _NC * _NS
_L = 16
_BLK = 128
_CB = 22
_CHUNK = _CB * _BLK
_R = 4


@functools.partial(jax.jit, static_argnames=("interpret",))
def _sc_partials(probs, edges, batch, interpret=False):
    N = probs.shape[0]
    E = edges.shape[1]
    nblk = E // _BLK
    mainb = nblk // _NW
    rem = nblk - mainb * _NW
    nchunk = mainb // _CB
    nvec = N // _L
    vpw = -(-nvec // _NW)

    mesh = plsc.VectorSubcoreMesh(
        core_axis_name="c", subcore_axis_name="s",
        num_cores=_NC, num_subcores=_NS)

    @functools.partial(
        pl.kernel,
        out_type=(
            jax.ShapeDtypeStruct((_NW, _L), jnp.float32),
            jax.ShapeDtypeStruct((_NW, _L), jnp.float32),
            jax.ShapeDtypeStruct((_NW, _L), jnp.int32),
        ),
        mesh=mesh,
        interpret=interpret,
        compiler_params=pltpu.CompilerParams(needs_layout_passes=False),
        scratch_types=[
            pltpu.VMEM((N,), jnp.float32),
            *[pltpu.VMEM((2, _CHUNK), jnp.int32) for _ in range(_R)],
            pltpu.VMEM((2, _BLK), jnp.int32),
            pltpu.VMEM((vpw * _L,), jnp.int32),
            pltpu.VMEM((_L,), jnp.float32),
            pltpu.VMEM((_L,), jnp.float32),
            pltpu.VMEM((_L,), jnp.int32),
            pltpu.SemaphoreType.DMA,
            pltpu.SemaphoreType.DMA,
            pltpu.SemaphoreType.DMA,
            *[pltpu.SemaphoreType.DMA for _ in range(_R)],
        ],
    )
    def sc_kernel(probs_hbm, edges_hbm, batch_hbm,
                  edge_out, psum_out, bmax_out,
                  table_v, *rest):
        ebufs = rest[:_R]
        xbuf, batch_v, oe_v, op_v, om_v, sem_t, sem_b, sem_x = rest[_R:_R + 8]
        esems = rest[_R + 8:]
        wid = lax.axis_index("s") * _NC + lax.axis_index("c")

        has_extra = wid < rem
        base_b = jnp.where(has_extra, wid * (mainb + 1), rem + wid * mainb)
        mstart = base_b + jnp.where(has_extra, 1, 0)

        def edge_cp(chunk, b):
            off = (mstart + chunk * _CB) * _BLK
            return pltpu.make_async_copy(
                edges_hbm.at[:, pl.ds(off, _CHUNK)], ebufs[b], esems[b])

        def edge_block(buf, off, a):
            ir = buf[0, pl.ds(off, _L)]
            ic = buf[1, pl.ds(off, _L)]
            pr = plsc.load_gather(table_v, [ir])
            pc = plsc.load_gather(table_v, [ic])
            return a + jnp.where(ir != ic, (1.0 - pr) * (1.0 - pc), 0.0)

        bstart = jnp.minimum(wid * vpw, nvec - vpw) * _L
        cp_t = pltpu.make_async_copy(probs_hbm, table_v, sem_t)
        cp_t.start()
        cp_b = pltpu.make_async_copy(
            batch_hbm.at[pl.ds(bstart, vpw * _L)], batch_v, sem_b)
        cp_b.start()
        for b in range(_R):
            edge_cp(b, b).start()

        cp_t.wait()
        pvec0 = wid * vpw
        ptrip = jnp.minimum(vpw, nvec - pvec0)

        @pl.loop(0, ptrip, init_carry=jnp.zeros((_L,), jnp.float32))
        def psum(i, acc):
            return acc + table_v[pl.ds((pvec0 + i) * _L, _L)]
        op_v[...] = psum

        cp_b.wait()

        @pl.loop(0, vpw, init_carry=jnp.zeros((_L,), jnp.int32))
        def bmax(i, acc):
            return jnp.maximum(acc, batch_v[pl.ds(i * _L, _L)])
        om_v[...] = bmax

        oe_v[...] = jnp.zeros((_L,), jnp.float32)

        @pl.when(has_extra)
        def _():
            cpx = pltpu.make_async_copy(
                edges_hbm.at[:, pl.ds(base_b * _BLK, _BLK)], xbuf, sem_x)
            cpx.start()
            cpx.wait()

            @pl.loop(0, _BLK, step=_L, init_carry=jnp.zeros((_L,), jnp.float32))
            def xacc(off, a):
                return edge_block(xbuf, off, a)
            oe_v[...] = xacc

        def chunk_compute(buf, acc):
            @plsc.parallel_loop(0, _CHUNK, step=_L, unroll=8, carry=acc)
            def inner(off, a):
                return edge_block(buf, off, a)
            return inner

        rounds = nchunk // _R

        @pl.loop(0, rounds, init_carry=oe_v[...])
        def eloop(g, acc):
            for b in range(_R):
                chunk = g * _R + b
                edge_cp(chunk, b).wait()
                acc = chunk_compute(ebufs[b], acc)

                refill = chunk + _R - 1
                cond = refill < nchunk
                if b == 0:
                    cond = jnp.logical_and(g > 0, cond)

                @pl.when(cond)
                def _():
                    edge_cp(refill, (b - 1) % _R).start()
            return acc

        acc = eloop
        for j in range(rounds * _R, nchunk):
            edge_cp(j, j % _R).wait()
            acc = chunk_compute(ebufs[j % _R], acc)
        oe_v[...] = acc

        pltpu.sync_copy(oe_v, edge_out.at[wid])
        pltpu.sync_copy(op_v, psum_out.at[wid])
        pltpu.sync_copy(om_v, bmax_out.at[wid])

    return sc_kernel(probs, edges, batch)


def kernel(probs, edge_index, batch, penalty_coefficient):
    e_part, p_part, m_part = _sc_partials(probs, edge_index, batch)
    num_graphs = jnp.max(m_part) + 1
    expected_distance = jnp.sum(e_part) / num_graphs
    expected_weight = jnp.sum(p_part) / num_graphs
    loss = penalty_coefficient * expected_distance + expected_weight
    return (loss, expected_weight, expected_distance)
